# Initial kernel scaffold; baseline (speedup 1.0000x reference)
import jax
import jax.numpy as jnp
from jax.experimental import pallas as pl


def kernel(x, router_w, router_b, ln1_g, ln1_b, qkv_w, qkv_b, proj_w, proj_b, ln2_g, ln2_b, l1_w, l1_b, l2_w, l2_b):
    raise NotImplementedError("write your pallas kernel here")


# speedup vs baseline: 2.3997x; 2.3997x over previous
import functools

import jax
import jax.numpy as jnp
from jax.experimental import pallas as pl
from jax.experimental.pallas import tpu as pltpu

E = 8
TOPK = 2
C = 768
H = 12
DH = C // H
N = 2048
HID = 3072
REG = 0.01
NT = 256
NTILES = N // NT


def _ln(x, g, b):
    m = jnp.mean(x, axis=-1, keepdims=True)
    xc = x - m
    v = jnp.mean(xc * xc, axis=-1, keepdims=True)
    return xc * jax.lax.rsqrt(v + 1e-5) * g + b


def _router_body(x_ref, w_ref, b_ref, noise_ref, w8_ref, bal_ref):
    x = x_ref[...]
    w = w_ref[...]
    logits = jax.lax.dot_general(x, w, (((1,), (1,)), ((), ())),
                                 preferred_element_type=jnp.float32)
    logits = logits + b_ref[...] + noise_ref[...]
    m = jnp.max(logits, axis=-1, keepdims=True)
    ex = jnp.exp(logits - m)
    probs = ex / jnp.sum(ex, axis=-1, keepdims=True)

    eidx = jax.lax.broadcasted_iota(jnp.int32, (1, E), 1)
    p1 = jnp.max(probs, axis=-1, keepdims=True)
    i1 = jnp.argmax(probs, axis=-1, keepdims=True)
    masked = jnp.where(eidx == i1, -jnp.inf, probs)
    p2 = jnp.max(masked, axis=-1, keepdims=True)
    i2 = jnp.argmax(masked, axis=-1, keepdims=True)
    s = p1 + p2
    w1 = p1 / s
    w2 = p2 / s
    onehot1 = (eidx == i1).astype(jnp.float32)
    onehot2 = (eidx == i2).astype(jnp.float32)
    w8_ref[...] = onehot1 * w1 + onehot2 * w2

    counts = jnp.sum(onehot1 + onehot2, axis=0)
    f_i = counts * (float(C) / float(N * TOPK))
    P_i = jnp.mean(probs, axis=0)
    rw_norm = jnp.sqrt(jnp.sum(w * w))
    bal_ref[0, 0] = float(E) * jnp.sum(f_i * P_i) + REG * rw_norm


def _router(x, router_w, router_b, noise):
    return pl.pallas_call(
        _router_body,
        out_shape=(
            jax.ShapeDtypeStruct((N, E), jnp.float32),
            jax.ShapeDtypeStruct((1, 1), jnp.float32),
        ),
        in_specs=[
            pl.BlockSpec((N, C), lambda: (0, 0)),
            pl.BlockSpec((E, C), lambda: (0, 0)),
            pl.BlockSpec((1, E), lambda: (0, 0)),
            pl.BlockSpec((N, E), lambda: (0, 0)),
        ],
        out_specs=(
            pl.BlockSpec((N, E), lambda: (0, 0)),
            pl.BlockSpec(memory_space=pltpu.SMEM),
        ),
    )(x, router_w, router_b.reshape(1, E), noise)


def _qkv_body(x_ref, g_ref, b_ref, w_ref, wb_ref, q_ref, k_ref, v_ref):
    h = _ln(x_ref[...], g_ref[0], b_ref[0])
    w = w_ref[0]
    out = jax.lax.dot_general(h, w, (((1,), (1,)), ((), ())),
                              preferred_element_type=jnp.float32)
    out = out + wb_ref[0]
    q_ref[0] = out[:, :C]
    k_ref[0] = out[:, C:2 * C]
    v_ref[0] = out[:, 2 * C:]


def _qkv(x, ln1_g, ln1_b, qkv_w, qkv_b):
    os = jax.ShapeDtypeStruct((E, N, C), jnp.float32)
    return pl.pallas_call(
        _qkv_body,
        grid=(E, NTILES),
        out_shape=(os, os, os),
        in_specs=[
            pl.BlockSpec((NT, C), lambda e, t: (t, 0)),
            pl.BlockSpec((1, 1, C), lambda e, t: (e, 0, 0)),
            pl.BlockSpec((1, 1, C), lambda e, t: (e, 0, 0)),
            pl.BlockSpec((1, 3 * C, C), lambda e, t: (e, 0, 0)),
            pl.BlockSpec((1, 1, 3 * C), lambda e, t: (e, 0, 0)),
        ],
        out_specs=tuple(
            pl.BlockSpec((1, NT, C), lambda e, t: (e, t, 0)) for _ in range(3)
        ),
    )(x, ln1_g.reshape(E, 1, C), ln1_b.reshape(E, 1, C), qkv_w,
      qkv_b.reshape(E, 1, 3 * C))


def _attn_body(q_ref, k_ref, v_ref, o_ref):
    q = q_ref[0]
    k = k_ref[0]
    v = v_ref[0]
    scale = float(DH) ** 0.5
    outs = []
    for h in range(H):
        sl = slice(h * DH, (h + 1) * DH)
        qh = q[:, sl]
        kh = k[:, sl]
        vh = v[:, sl]
        s = jax.lax.dot_general(qh, kh, (((1,), (1,)), ((), ())),
                                preferred_element_type=jnp.float32) * scale
        s = s - jnp.max(s, axis=-1, keepdims=True)
        p = jnp.exp(s)
        p = p / jnp.sum(p, axis=-1, keepdims=True)
        outs.append(jnp.dot(p, vh, preferred_element_type=jnp.float32))
    o_ref[0] = jnp.concatenate(outs, axis=-1)


def _attn(q, k, v):
    return pl.pallas_call(
        _attn_body,
        grid=(E, NTILES),
        out_shape=jax.ShapeDtypeStruct((E, N, C), jnp.float32),
        in_specs=[
            pl.BlockSpec((1, NT, C), lambda e, t: (e, t, 0)),
            pl.BlockSpec((1, N, C), lambda e, t: (e, 0, 0)),
            pl.BlockSpec((1, N, C), lambda e, t: (e, 0, 0)),
        ],
        out_specs=pl.BlockSpec((1, NT, C), lambda e, t: (e, t, 0)),
    )(q, k, v)


def _mlp_body(x_ref, o_ref, pw_ref, pb_ref, g2_ref, b2_ref,
              w1_ref, b1_ref, w2_ref, b2b_ref, eo_ref):
    x = x_ref[...]
    o = o_ref[0]
    op = jax.lax.dot_general(o, pw_ref[0], (((1,), (1,)), ((), ())),
                             preferred_element_type=jnp.float32) + pb_ref[0]
    x1 = x + op
    h2 = _ln(x1, g2_ref[0], b2_ref[0])
    a = jax.lax.dot_general(h2, w1_ref[0], (((1,), (1,)), ((), ())),
                            preferred_element_type=jnp.float32) + b1_ref[0]
    a = 0.5 * a * (1.0 + jax.lax.erf(a * (2.0 ** -0.5)))
    out = jax.lax.dot_general(a, w2_ref[0], (((1,), (1,)), ((), ())),
                              preferred_element_type=jnp.float32) + b2b_ref[0]
    eo_ref[0] = x1 + out


def _mlp(x, o, proj_w, proj_b, ln2_g, ln2_b, l1_w, l1_b, l2_w, l2_b):
    return pl.pallas_call(
        _mlp_body,
        grid=(E, NTILES),
        out_shape=jax.ShapeDtypeStruct((E, N, C), jnp.float32),
        in_specs=[
            pl.BlockSpec((NT, C), lambda e, t: (t, 0)),
            pl.BlockSpec((1, NT, C), lambda e, t: (e, t, 0)),
            pl.BlockSpec((1, C, C), lambda e, t: (e, 0, 0)),
            pl.BlockSpec((1, 1, C), lambda e, t: (e, 0, 0)),
            pl.BlockSpec((1, 1, C), lambda e, t: (e, 0, 0)),
            pl.BlockSpec((1, 1, C), lambda e, t: (e, 0, 0)),
            pl.BlockSpec((1, HID, C), lambda e, t: (e, 0, 0)),
            pl.BlockSpec((1, 1, HID), lambda e, t: (e, 0, 0)),
            pl.BlockSpec((1, C, HID), lambda e, t: (e, 0, 0)),
            pl.BlockSpec((1, 1, C), lambda e, t: (e, 0, 0)),
        ],
        out_specs=pl.BlockSpec((1, NT, C), lambda e, t: (e, t, 0)),
    )(x, o, proj_w, proj_b.reshape(E, 1, C), ln2_g.reshape(E, 1, C),
      ln2_b.reshape(E, 1, C), l1_w, l1_b.reshape(E, 1, HID), l2_w,
      l2_b.reshape(E, 1, C))


def _combine_body(eo_ref, w8_ref, out_ref):
    acc = eo_ref[0] * w8_ref[:, 0:1]
    for e in range(1, E):
        acc = acc + eo_ref[e] * w8_ref[:, e:e + 1]
    out_ref[...] = acc


def _combine(eo, w8):
    return pl.pallas_call(
        _combine_body,
        grid=(NTILES,),
        out_shape=jax.ShapeDtypeStruct((N, C), jnp.float32),
        in_specs=[
            pl.BlockSpec((E, NT, C), lambda t: (0, t, 0)),
            pl.BlockSpec((NT, E), lambda t: (t, 0)),
        ],
        out_specs=pl.BlockSpec((NT, C), lambda t: (t, 0)),
    )(eo, w8)


def kernel(x, router_w, router_b, ln1_g, ln1_b, qkv_w, qkv_b, proj_w, proj_b,
           ln2_g, ln2_b, l1_w, l1_b, l2_w, l2_b):
    xb = x.reshape(N, C)
    noise = (jax.random.uniform(jax.random.key(42), (1, N, E), jnp.float32)
             * 0.1).reshape(N, E)
    w8, bal = _router(xb, router_w, router_b, noise)
    q, k, v = _qkv(xb, ln1_g, ln1_b, qkv_w, qkv_b)
    o = _attn(q, k, v)
    eo = _mlp(xb, o, proj_w, proj_b, ln2_g, ln2_b, l1_w, l1_b, l2_w, l2_b)
    combine = _combine(eo, w8)
    return combine.reshape(1, N, C), bal[0, 0]
